# baseline (speedup 1.0000x reference)
import functools

import jax
import jax.numpy as jnp
from jax.experimental import pallas as pl
from jax.experimental.pallas import tpu as pltpu


def _fused_kernel(ep_ref, x_ref, colmask_ref, seg_ref, tile_ref,
                  wc1j_ref, wc2j_ref, w0_ref, b0j_ref, w1j_ref, b1j_ref,
                  out_ref, ext2_ref, *, S, W, bt, Ce):
    f32 = jnp.float32
    F = w1j_ref.shape[0] // 2
    C0 = wc2j_ref.shape[0] // 18
    SW = S * W

    wc1j = wc1j_ref[...]
    wc2j = wc2j_ref[...]
    w0 = w0_ref[...]
    w1j = w1j_ref[...]
    b0j = b0j_ref[...]
    b1j = b1j_ref[...]
    colmask = colmask_ref[...]
    seg = seg_ref[...]
    tile = tile_ref[...]

    pad = W + 1
    shifts = [(dy - 1) * W + (dx - 1) for dy in range(3) for dx in range(3)]

    @pl.when(pl.program_id(0) == 0)
    def _init():
        ext2_ref[0:pad, :] = jnp.zeros((pad, 2 * C0), f32)
        ext2_ref[pad + SW:, :] = jnp.zeros((pad, 2 * C0), f32)

    a = ep_ref[0]
    cols1 = jnp.concatenate(
        [a[pad + d:pad + d + SW, :] for d in shifts], axis=-1)
    e1a = jnp.dot(cols1, wc1j, preferred_element_type=f32)
    e1m = e1a * colmask

    for p in range(bt // 2):
        ext2_ref[pad:pad + SW, :] = e1m[:, 2 * p * C0:2 * (p + 1) * C0]
        cols2 = jnp.concatenate(
            [ext2_ref[pad + d:pad + d + SW, :] for d in shifts], axis=-1)
        ee = jnp.dot(cols2, wc2j, preferred_element_type=f32)

        s0a = jnp.dot(x_ref[2 * p], w0, preferred_element_type=f32)
        s0b = jnp.dot(x_ref[2 * p + 1], w0, preferred_element_type=f32)
        s0 = jnp.concatenate([s0a, s0b], axis=1)
        s0t = jnp.dot(tile, s0, preferred_element_type=f32)
        out0 = jnp.dot(seg, ee * s0t, preferred_element_type=f32) + b0j
        s1 = jnp.dot(out0, w1j, preferred_element_type=f32)
        s1t = jnp.dot(tile, s1, preferred_element_type=f32)
        out1 = jnp.dot(seg, ee * s1t, preferred_element_type=f32) + b1j
        out_ref[2 * p] = out1[:, :F].astype(out_ref.dtype)
        out_ref[2 * p + 1] = out1[:, F:].astype(out_ref.dtype)


@functools.partial(jax.jit, static_argnames=("batch_tile",))
def _run(x, edge, conv1_w, conv2_w, w0, b0, w1, b1, batch_tile=8):
    f32 = jnp.float32
    B, S, Fn = x.shape
    Ce = edge.shape[-1]
    C0 = conv1_w.shape[0]
    C1 = conv2_w.shape[0]
    F = C0 + C1
    W = S + 2
    SW = S * W
    pad = W + 1
    R = SW + 2 * pad
    bt = batch_tile
    G = B // bt

    xf = x.astype(f32)
    Cp = 8
    ep = jnp.pad(edge.astype(f32),
                 ((0, 0), (0, 0), (0, W - S), (0, Cp - Ce))).reshape(
                     B, SW, Cp)
    ep = jnp.pad(ep, ((0, 0), (pad, pad), (0, 0)))
    ep = ep.reshape(G, bt, R, Cp).transpose(0, 2, 1, 3).reshape(G, R, bt * Cp)

    wc1 = jnp.transpose(conv1_w, (2, 3, 1, 0)).reshape(9, Ce, C0).astype(f32)
    wc1 = jnp.pad(wc1, ((0, 0), (0, Cp - Ce), (0, 0)))
    wc1j = jnp.einsum('kco,bd->kbcdo', wc1, jnp.eye(bt, dtype=f32))
    wc1j = wc1j.reshape(9 * bt * Cp, bt * C0)
    wc2 = jnp.transpose(conv2_w, (2, 3, 1, 0)).reshape(9 * C0, C1).astype(f32)
    eye_center = jnp.zeros((9 * C0, C0), f32).at[4 * C0 + jnp.arange(C0),
                                                 jnp.arange(C0)].set(1.0)
    wc2a = jnp.concatenate([eye_center, wc2], axis=1)
    wc2j = jnp.einsum('kco,bd->kbcdo', wc2a.reshape(9, C0, F),
                      jnp.eye(2, dtype=f32)).reshape(18 * C0, 2 * F)

    w0f = w0.astype(f32)
    w1j = jnp.kron(jnp.eye(2, dtype=f32), w1.astype(f32))
    b0j = jnp.tile(b0.reshape(1, F).astype(f32), (1, 2))
    b1j = jnp.tile(b1.reshape(1, F).astype(f32), (1, 2))
    rr = jnp.arange(SW)
    colmask = jnp.broadcast_to(((rr % W) < S).astype(f32)[:, None],
                               (SW, bt * C0))
    seg = (rr[None, :] // W == jnp.arange(S)[:, None]).astype(f32)
    tile = (rr[:, None] % W == jnp.arange(S)[None, :]).astype(f32)

    def const_spec(shape):
        z = (0,) * len(shape)
        return pl.BlockSpec(shape, lambda g, _z=z: _z)

    flops = 2 * B * (SW * (9 * Ce) * C0 + SW * (9 * C0) * F
                     + S * Fn * F + S * F * F
                     + 2 * (SW * S * F + S * SW * F + SW * F))
    bytes_accessed = 4 * (ep.size + xf.size + colmask.size + seg.size
                          + tile.size + wc1j.size + wc2j.size + w0f.size
                          + w1j.size + b0j.size + b1j.size + B * S * F)

    return pl.pallas_call(
        functools.partial(_fused_kernel, S=S, W=W, bt=bt, Ce=Ce),
        grid=(G,),
        in_specs=[
            pl.BlockSpec((1, R, bt * Cp), lambda g: (g, 0, 0)),
            pl.BlockSpec((bt, S, Fn), lambda g: (g, 0, 0)),
            const_spec((SW, bt * C0)),
            const_spec((S, SW)),
            const_spec((SW, S)),
            const_spec((9 * bt * Cp, bt * C0)),
            const_spec((18 * C0, 2 * F)),
            const_spec((Fn, F)), const_spec((1, 2 * F)),
            const_spec((2 * F, 2 * F)), const_spec((1, 2 * F)),
        ],
        out_specs=pl.BlockSpec((bt, S, F), lambda g: (g, 0, 0)),
        out_shape=jax.ShapeDtypeStruct((B, S, F), f32),
        scratch_shapes=[pltpu.VMEM((R, 2 * C0), f32)],
        compiler_params=pltpu.CompilerParams(dimension_semantics=("parallel",)),
        cost_estimate=pl.CostEstimate(flops=flops, transcendentals=0,
                                      bytes_accessed=bytes_accessed),
    )(ep, xf, colmask, seg, tile, wc1j, wc2j, w0f, b0j, w1j, b1j)


def kernel(x, edge, conv1_w, conv2_w, w0, b0, w1, b1):
    return _run(x, edge, conv1_w, conv2_w, w0, b0, w1, b1)


# speedup vs baseline: 1.0137x; 1.0137x over previous
import functools

import jax
import jax.numpy as jnp
from jax.experimental import pallas as pl
from jax.experimental.pallas import tpu as pltpu


def _fused_kernel(ep_ref, x_ref, colmask_ref, seg_ref, tile_ref,
                  wc1j_ref, wc2a_ref, w0_ref, b0_ref, w1_ref, b1_ref,
                  out_ref, ext2_ref, *, S, W, bt, Ce):
    f32 = jnp.float32
    C0 = wc2a_ref.shape[0] // 9
    F = wc2a_ref.shape[1]
    SW = S * W

    wc1j = wc1j_ref[...]
    wc2a = wc2a_ref[...]
    w0 = w0_ref[...]
    w1 = w1_ref[...]
    b0 = b0_ref[...]
    b1 = b1_ref[...]
    colmask = colmask_ref[...]
    seg = seg_ref[...]
    tile = tile_ref[...]

    pad = W + 1
    shifts = [(dy - 1) * W + (dx - 1) for dy in range(3) for dx in range(3)]

    @pl.when(pl.program_id(0) == 0)
    def _init():
        ext2_ref[0:pad, :] = jnp.zeros((pad, C0), f32)
        ext2_ref[pad + SW:, :] = jnp.zeros((pad, C0), f32)

    a = ep_ref[0]
    cols1 = jnp.concatenate(
        [a[pad + d:pad + d + SW, :] for d in shifts], axis=-1)
    e1a = jnp.dot(cols1, wc1j, preferred_element_type=f32)
    e1m = e1a * colmask

    for b in range(bt):
        ext2_ref[pad:pad + SW, :] = e1m[:, b * C0:(b + 1) * C0]
        cols2 = jnp.concatenate(
            [ext2_ref[pad + d:pad + d + SW, :] for d in shifts], axis=-1)
        e = jnp.dot(cols2, wc2a, preferred_element_type=f32)

        s0 = jnp.dot(x_ref[b], w0, preferred_element_type=f32)
        s0t = jnp.dot(tile, s0, preferred_element_type=f32)
        out0 = jnp.dot(seg, e * s0t, preferred_element_type=f32) + b0
        s1 = jnp.dot(out0, w1, preferred_element_type=f32)
        s1t = jnp.dot(tile, s1, preferred_element_type=f32)
        out1 = jnp.dot(seg, e * s1t, preferred_element_type=f32) + b1
        out_ref[b] = out1.astype(out_ref.dtype)


@functools.partial(jax.jit, static_argnames=("batch_tile",))
def _run(x, edge, conv1_w, conv2_w, w0, b0, w1, b1, batch_tile=8):
    f32 = jnp.float32
    B, S, Fn = x.shape
    Ce = edge.shape[-1]
    C0 = conv1_w.shape[0]
    C1 = conv2_w.shape[0]
    F = C0 + C1
    W = S + 2
    SW = S * W
    pad = W + 1
    R = SW + 2 * pad
    bt = batch_tile
    G = B // bt

    xf = x.astype(f32)
    Cp = 8
    ep = jnp.pad(edge.astype(f32),
                 ((0, 0), (0, 0), (0, W - S), (0, Cp - Ce))).reshape(
                     B, SW, Cp)
    ep = jnp.pad(ep, ((0, 0), (pad, pad), (0, 0)))
    ep = ep.reshape(G, bt, R, Cp).transpose(0, 2, 1, 3).reshape(G, R, bt * Cp)

    wc1 = jnp.transpose(conv1_w, (2, 3, 1, 0)).reshape(9, Ce, C0).astype(f32)
    wc1 = jnp.pad(wc1, ((0, 0), (0, Cp - Ce), (0, 0)))
    wc1j = jnp.einsum('kco,bd->kbcdo', wc1, jnp.eye(bt, dtype=f32))
    wc1j = wc1j.reshape(9 * bt * Cp, bt * C0)
    wc2 = jnp.transpose(conv2_w, (2, 3, 1, 0)).reshape(9 * C0, C1).astype(f32)
    eye_center = jnp.zeros((9 * C0, C0), f32).at[4 * C0 + jnp.arange(C0),
                                                 jnp.arange(C0)].set(1.0)
    wc2a = jnp.concatenate([eye_center, wc2], axis=1)

    w0f = w0.astype(f32)
    w1f = w1.astype(f32)
    b0f = b0.reshape(1, F).astype(f32)
    b1f = b1.reshape(1, F).astype(f32)
    rr = jnp.arange(SW)
    colmask = jnp.broadcast_to(((rr % W) < S).astype(f32)[:, None],
                               (SW, bt * C0))
    seg = (rr[None, :] // W == jnp.arange(S)[:, None]).astype(f32)
    tile = (rr[:, None] % W == jnp.arange(S)[None, :]).astype(f32)

    def const_spec(shape):
        z = (0,) * len(shape)
        return pl.BlockSpec(shape, lambda g, _z=z: _z)

    flops = 2 * B * (SW * (9 * Ce) * C0 + SW * (9 * C0) * F
                     + S * Fn * F + S * F * F
                     + 2 * (SW * S * F + S * SW * F + SW * F))
    bytes_accessed = 4 * (ep.size + xf.size + colmask.size + seg.size
                          + tile.size + wc1j.size + wc2a.size + w0f.size
                          + w1f.size + b0f.size + b1f.size + B * S * F)

    return pl.pallas_call(
        functools.partial(_fused_kernel, S=S, W=W, bt=bt, Ce=Ce),
        grid=(G,),
        in_specs=[
            pl.BlockSpec((1, R, bt * Cp), lambda g: (g, 0, 0)),
            pl.BlockSpec((bt, S, Fn), lambda g: (g, 0, 0)),
            const_spec((SW, bt * C0)),
            const_spec((S, SW)),
            const_spec((SW, S)),
            const_spec((9 * bt * Cp, bt * C0)),
            const_spec((9 * C0, F)),
            const_spec((Fn, F)), const_spec((1, F)),
            const_spec((F, F)), const_spec((1, F)),
        ],
        out_specs=pl.BlockSpec((bt, S, F), lambda g: (g, 0, 0)),
        out_shape=jax.ShapeDtypeStruct((B, S, F), f32),
        scratch_shapes=[pltpu.VMEM((R, C0), f32)],
        compiler_params=pltpu.CompilerParams(dimension_semantics=("parallel",)),
        cost_estimate=pl.CostEstimate(flops=flops, transcendentals=0,
                                      bytes_accessed=bytes_accessed),
    )(ep, xf, colmask, seg, tile, wc1j, wc2a, w0f, b0f, w1f, b1f)


def kernel(x, edge, conv1_w, conv2_w, w0, b0, w1, b1):
    return _run(x, edge, conv1_w, conv2_w, w0, b0, w1, b1)
